# baseline (device time: 10167 ns/iter reference)
import jax
import jax.numpy as jnp
from jax import lax
from jax.experimental import pallas as pl
from jax.experimental.pallas import tpu as pltpu

N_DEV = 4
N_C = 16


def kernel(t):
    m, n = t.shape
    c = m // N_C
    half = N_C // 2

    def body(x_ref, out_ref, comm_ref, send_sems, recv_sems):
        p = lax.axis_index("i")
        pa = p ^ 1
        pb = 3 - p
        part1 = [pa if i < half else pb for i in range(N_C)]
        part2 = [pb if i < half else pa for i in range(N_C)]
        order = [k + j * half for k in range(half) for j in (0, 1)]

        def mk(src_slot, dst_slot, sem, dev):
            return pltpu.make_async_remote_copy(
                src_ref=comm_ref.at[src_slot],
                dst_ref=comm_ref.at[dst_slot],
                send_sem=send_sems.at[sem],
                recv_sem=recv_sems.at[sem],
                device_id=(dev,),
                device_id_type=pl.DeviceIdType.MESH,
            )

        r1 = [mk(i, N_C + i, i, part1[i]) for i in range(N_C)]
        r2 = [mk(2 * N_C + i, 3 * N_C + i, N_C + i, part2[i]) for i in range(N_C)]

        barrier_sem = pltpu.get_barrier_semaphore()
        for prt in [pa, pb]:
            pl.semaphore_signal(
                barrier_sem, inc=1,
                device_id=(prt,), device_id_type=pl.DeviceIdType.MESH,
            )
        comm_ref[order[0], :, :] = x_ref[
            order[0] * c : (order[0] + 1) * c, :
        ].astype(jnp.bfloat16)
        comm_ref[order[1], :, :] = x_ref[
            order[1] * c : (order[1] + 1) * c, :
        ].astype(jnp.bfloat16)
        pl.semaphore_wait(barrier_sem, 2)

        r1[order[0]].start()
        r1[order[1]].start()
        for i in order[2:]:
            comm_ref[i, :, :] = x_ref[i * c : (i + 1) * c, :].astype(jnp.bfloat16)
            r1[i].start()

        for i in order:
            r1[i].wait_recv()
            comm_ref[2 * N_C + i, :, :] = (
                comm_ref[i, :, :] + comm_ref[N_C + i, :, :]
            )
            r2[i].start()

        for i in order:
            r2[i].wait_recv()
            s = (
                comm_ref[2 * N_C + i, :, :] + comm_ref[3 * N_C + i, :, :]
            ).astype(jnp.float32)
            r = jnp.maximum(s, 0.0)
            out_ref[i * c : (i + 1) * c, :] = jnp.tanh(s) * s * s + r * r * r

        for i in range(N_C):
            r1[i].wait_send()
            r2[i].wait_send()

    return pl.pallas_call(
        body,
        out_shape=jax.ShapeDtypeStruct((m, n), jnp.float32),
        in_specs=[pl.BlockSpec(memory_space=pltpu.VMEM)],
        out_specs=pl.BlockSpec(memory_space=pltpu.VMEM),
        scratch_shapes=[
            pltpu.VMEM((4 * N_C, c, n), jnp.bfloat16),
            pltpu.SemaphoreType.DMA((2 * N_C,)),
            pltpu.SemaphoreType.DMA((2 * N_C,)),
        ],
        compiler_params=pltpu.CompilerParams(collective_id=0),
    )(t)


# device time: 10017 ns/iter; 1.0150x vs baseline; 1.0150x over previous
import jax
import jax.numpy as jnp
from jax import lax
from jax.experimental import pallas as pl
from jax.experimental.pallas import tpu as pltpu

N_DEV = 4
N_C = 8


def kernel(t):
    m, n = t.shape
    c = m // N_C
    half = N_C // 2

    def body(x_ref, out_ref, comm_ref, send_sems, recv_sems):
        p = lax.axis_index("i")
        pa = p ^ 1
        pb = 3 - p
        part1 = [pa if i < half else pb for i in range(N_C)]
        part2 = [pb if i < half else pa for i in range(N_C)]
        order = [k + j * half for k in range(half) for j in (0, 1)]

        def mk(src_slot, dst_slot, sem, dev):
            return pltpu.make_async_remote_copy(
                src_ref=comm_ref.at[src_slot],
                dst_ref=comm_ref.at[dst_slot],
                send_sem=send_sems.at[sem],
                recv_sem=recv_sems.at[sem],
                device_id=(dev,),
                device_id_type=pl.DeviceIdType.MESH,
            )

        r1 = [mk(i, N_C + i, i, part1[i]) for i in range(N_C)]
        r2 = [mk(2 * N_C + i, 3 * N_C + i, N_C + i, part2[i]) for i in range(N_C)]

        barrier_sem = pltpu.get_barrier_semaphore()
        for prt in [pa, pb]:
            pl.semaphore_signal(
                barrier_sem, inc=1,
                device_id=(prt,), device_id_type=pl.DeviceIdType.MESH,
            )
        comm_ref[order[0], :, :] = x_ref[
            order[0] * c : (order[0] + 1) * c, :
        ].astype(jnp.bfloat16)
        comm_ref[order[1], :, :] = x_ref[
            order[1] * c : (order[1] + 1) * c, :
        ].astype(jnp.bfloat16)
        pl.semaphore_wait(barrier_sem, 2)

        r1[order[0]].start()
        r1[order[1]].start()
        for i in order[2:]:
            comm_ref[i, :, :] = x_ref[i * c : (i + 1) * c, :].astype(jnp.bfloat16)
            r1[i].start()

        for i in order:
            r1[i].wait_recv()
            comm_ref[2 * N_C + i, :, :] = (
                comm_ref[i, :, :] + comm_ref[N_C + i, :, :]
            )
            r2[i].start()

        for i in order:
            r2[i].wait_recv()
            s = (
                comm_ref[2 * N_C + i, :, :] + comm_ref[3 * N_C + i, :, :]
            ).astype(jnp.float32)
            r = jnp.maximum(s, 0.0)
            out_ref[i * c : (i + 1) * c, :] = jnp.tanh(s) * s * s + r * r * r

        for i in range(N_C):
            r1[i].wait_send()
            r2[i].wait_send()

    return pl.pallas_call(
        body,
        out_shape=jax.ShapeDtypeStruct((m, n), jnp.float32),
        in_specs=[pl.BlockSpec(memory_space=pltpu.VMEM)],
        out_specs=pl.BlockSpec(memory_space=pltpu.VMEM),
        scratch_shapes=[
            pltpu.VMEM((4 * N_C, c, n), jnp.bfloat16),
            pltpu.SemaphoreType.DMA((2 * N_C,)),
            pltpu.SemaphoreType.DMA((2 * N_C,)),
        ],
        compiler_params=pltpu.CompilerParams(collective_id=0),
    )(t)
